# Initial kernel scaffold; baseline (speedup 1.0000x reference)
import jax
import jax.numpy as jnp
from jax.experimental import pallas as pl


def kernel(x, edge_index, W, b, gamma, beta):
    raise NotImplementedError("write your pallas kernel here")


# speedup vs baseline: 13.4434x; 13.4434x over previous
import jax
import jax.numpy as jnp
from jax import lax
from jax.experimental import pallas as pl
from jax.experimental.pallas import tpu as pltpu
from jax.experimental.pallas import tpu_sc as plsc

NC = 2
NS = 16
NW = NC * NS
BK = 128


def _nan_guard(v):
    return jnp.nan_to_num(v, nan=0.0)


def kernel(x, edge_index, W, b, gamma, beta):
    N, D = x.shape
    E = edge_index.shape[1]
    EPW = -(-E // NW)
    NB = -(-EPW // BK)
    EPAD = NW * NB * BK
    NP = (-(-(N + 1) // BK)) * BK
    NPT = NP // NS
    assert NPT % 8 == 0

    src = edge_index[0].astype(jnp.int32)
    dst = edge_index[1].astype(jnp.int32)
    pad = EPAD - E
    src3 = jnp.concatenate([src, jnp.zeros((pad,), jnp.int32)]).reshape(NW, NB, BK)
    dst3 = jnp.concatenate([dst, jnp.full((pad,), N, jnp.int32)]).reshape(NW, NB, BK)

    zrow = jnp.zeros((NPT,), jnp.float32)
    zacc = jnp.zeros((BK, D), jnp.float32)
    ones = jnp.ones((BK,), jnp.float32)

    mesh = plsc.VectorSubcoreMesh(core_axis_name="c", subcore_axis_name="s")

    def deg_body(dst_hbm, ones_hbm, zrow_hbm, deg_hbm, idx_v, ones_v, stage_v,
                 acc_sh):
        c = lax.axis_index("c")
        s = lax.axis_index("s")
        w = c * NS + s
        pltpu.sync_copy(zrow_hbm, stage_v)
        pltpu.sync_copy(stage_v, acc_sh.at[pl.ds(s * NPT, NPT)])
        pltpu.sync_copy(ones_hbm, ones_v)
        pltpu.sync_copy(dst_hbm.at[w], idx_v)
        plsc.subcore_barrier()

        def blk(j, carry):
            pltpu.sync_copy(ones_v, acc_sh.at[idx_v.at[j]], add=True)
            return carry

        lax.fori_loop(0, NB, blk, 0)
        plsc.subcore_barrier()
        pltpu.sync_copy(acc_sh.at[pl.ds(s * NPT, NPT)], stage_v)
        pltpu.sync_copy(stage_v, deg_hbm.at[pl.ds(c * NP + s * NPT, NPT)])

    deg_1d = pl.kernel(
        deg_body,
        out_type=jax.ShapeDtypeStruct((NC * NP,), jnp.float32),
        mesh=mesh,
        scratch_types=[
            pltpu.VMEM((NB, BK), jnp.int32),
            pltpu.VMEM((BK,), jnp.float32),
            pltpu.VMEM((NPT,), jnp.float32),
            pltpu.VMEM_SHARED((NP,), jnp.float32),
        ],
    )(dst3, ones, zrow)
    deg_p = deg_1d.reshape(NC, NP)

    xp = jnp.pad(x, ((0, NP - N), (0, 0)))
    BM = 128

    def mm_body(x_ref, w_ref, o_ref):
        o_ref[...] = jnp.dot(_nan_guard(x_ref[...]), w_ref[...],
                             preferred_element_type=jnp.float32)

    xw = pl.pallas_call(
        mm_body,
        grid=(NP // BM,),
        in_specs=[
            pl.BlockSpec((BM, D), lambda i: (i, 0)),
            pl.BlockSpec((D, D), lambda i: (0, 0)),
        ],
        out_specs=pl.BlockSpec((BM, D), lambda i: (i, 0)),
        out_shape=jax.ShapeDtypeStruct((NP, D), jnp.float32),
    )(xp, W)

    dpt = deg_p.T

    def yw_body(xw_ref, dpt_ref, yw_ref, dv_ref):
        deg = dpt_ref[:, 0:1] + dpt_ref[:, 1:2] + 1.0
        dv = lax.rsqrt(deg)
        dv_ref[...] = dv
        yw_ref[...] = xw_ref[...] * dv

    yw, dinv = pl.pallas_call(
        yw_body,
        grid=(NP // BM,),
        in_specs=[
            pl.BlockSpec((BM, D), lambda i: (i, 0)),
            pl.BlockSpec((BM, NC), lambda i: (i, 0)),
        ],
        out_specs=[
            pl.BlockSpec((BM, D), lambda i: (i, 0)),
            pl.BlockSpec((BM, 1), lambda i: (i, 0)),
        ],
        out_shape=[
            jax.ShapeDtypeStruct((NP, D), jnp.float32),
            jax.ShapeDtypeStruct((NP, 1), jnp.float32),
        ],
    )(xw, dpt)

    NKF = NPT // BK
    TAIL = NPT - NKF * BK

    def agg_body(src_hbm, dst_hbm, yw_hbm, zacc_hbm, acc_hbm,
                 sidx_v, didx_v, rows_v, acc_sh):
        c = lax.axis_index("c")
        s = lax.axis_index("s")
        w = c * NS + s
        pltpu.sync_copy(zacc_hbm, rows_v)
        for k in range(NKF):
            pltpu.sync_copy(rows_v, acc_sh.at[pl.ds(s * NPT + k * BK, BK)])
        if TAIL:
            pltpu.sync_copy(rows_v.at[pl.ds(0, TAIL)],
                            acc_sh.at[pl.ds(s * NPT + NKF * BK, TAIL)])
        pltpu.sync_copy(src_hbm.at[w], sidx_v)
        pltpu.sync_copy(dst_hbm.at[w], didx_v)
        plsc.subcore_barrier()

        def blk(j, carry):
            pltpu.sync_copy(yw_hbm.at[sidx_v.at[j]], rows_v)
            pltpu.sync_copy(rows_v, acc_sh.at[didx_v.at[j]], add=True)
            return carry

        lax.fori_loop(0, NB, blk, 0)
        plsc.subcore_barrier()
        for k in range(NKF):
            pltpu.sync_copy(acc_sh.at[pl.ds(s * NPT + k * BK, BK)], rows_v)
            pltpu.sync_copy(rows_v, acc_hbm.at[c, pl.ds(s * NPT + k * BK, BK)])
        if TAIL:
            pltpu.sync_copy(acc_sh.at[pl.ds(s * NPT + NKF * BK, TAIL)],
                            rows_v.at[pl.ds(0, TAIL)])
            pltpu.sync_copy(rows_v.at[pl.ds(0, TAIL)],
                            acc_hbm.at[c, pl.ds(s * NPT + NKF * BK, TAIL)])

    acc = pl.kernel(
        agg_body,
        out_type=jax.ShapeDtypeStruct((NC, NP, D), jnp.float32),
        mesh=mesh,
        scratch_types=[
            pltpu.VMEM((NB, BK), jnp.int32),
            pltpu.VMEM((NB, BK), jnp.int32),
            pltpu.VMEM((BK, D), jnp.float32),
            pltpu.VMEM_SHARED((NP, D), jnp.float32),
        ],
    )(src3, dst3, yw, zacc)

    a0 = acc[0, :N]
    a1 = acc[1, :N]
    ywn = yw[:N]
    dvn = dinv[:N]
    b2 = b.reshape(1, D)
    g2 = gamma.reshape(1, D)
    be2 = beta.reshape(1, D)

    def fin_body(a0_ref, a1_ref, yw_ref, dv_ref, x_ref, b_ref, g_ref, be_ref,
                 o_ref):
        pre = (a0_ref[...] + a1_ref[...] + yw_ref[...]) * dv_ref[...] + b_ref[...]
        m = jnp.mean(pre, axis=0, keepdims=True)
        v = jnp.mean((pre - m) * (pre - m), axis=0, keepdims=True)
        o = (pre - m) * (g_ref[...] * lax.rsqrt(v + 1e-5)) + be_ref[...]
        o = jnp.maximum(o, 0.0) + _nan_guard(x_ref[...])
        o_ref[...] = _nan_guard(o)

    out = pl.pallas_call(
        fin_body,
        out_shape=jax.ShapeDtypeStruct((N, D), jnp.float32),
    )(a0, a1, ywn, dvn, x, b2, g2, be2)

    return out
